# baseline (speedup 1.0000x reference)
import functools

import jax
import jax.numpy as jnp
from jax import lax
from jax.experimental import pallas as pl
from jax.experimental.pallas import tpu as pltpu
from jax.experimental.pallas import tpu_sc as plsc

_B, _N = 8, 2048
_L = 16
_NC, _NS = 2, 16
_NW = _NC * _NS
_RPW = _B * _N // _NW
_WPB = _NW // _B
_GRP = _RPW // _L
_CHUNKS = _N // _L
_TR = 256
_NT = _N // _TR
_BIG = float(3.0e38)

_RADIUS = 0.07
_H = 0.03
_EPS = 1e-12


def _sc_body(pred_hbm, bpred_hbm, out_tk, out_tv, pco, bpc, qnb, tkb, tvb):
    wid = lax.axis_index("s") * _NC + lax.axis_index("c")
    b = wid // _WPB
    base = (wid % _WPB) * _RPW

    pltpu.sync_copy(pred_hbm.at[b], pco)
    pltpu.sync_copy(bpred_hbm.at[b], bpc)

    def prep(c, carry):
        s = pl.ds(c * _L, _L)
        qx = pco[s]
        qy = pco[pl.ds(_N + c * _L, _L)]
        qz = pco[pl.ds(2 * _N + c * _L, _L)]
        qnb[s] = qx * qx + qy * qy + qz * qz
        return carry

    lax.fori_loop(0, _CHUNKS, prep, 0)

    def grp16(g, carry):
        rbase = base + g * _L
        pxc = pco[pl.ds(rbase, _L)]
        pyc = pco[pl.ds(_N + rbase, _L)]
        pzc = pco[pl.ds(2 * _N + rbase, _L)]
        bxc = bpc[pl.ds(rbase, _L)]
        byc = bpc[pl.ds(_N + rbase, _L)]
        bzc = bpc[pl.ds(2 * _N + rbase, _L)]
        pnc = qnb[pl.ds(rbase, _L)]

        for pair in range(_L // 2):
            row = []
            for i in range(2):
                l = pair * 2 + i
                row.append((
                    jnp.broadcast_to(pxc[l], (_L,)),
                    jnp.broadcast_to(pyc[l], (_L,)),
                    jnp.broadcast_to(pzc[l], (_L,)),
                    jnp.broadcast_to(bxc[l], (_L,)),
                    jnp.broadcast_to(byc[l], (_L,)),
                    jnp.broadcast_to(bzc[l], (_L,)),
                    jnp.broadcast_to(pnc[l], (_L,)),
                ))

            init = (jnp.full((_L,), _BIG, jnp.float32),) * 20

            def chunk(c, st, row=row):
                st = list(st)
                s0 = pl.ds(c * _L, _L)
                s1 = pl.ds(_N + c * _L, _L)
                s2 = pl.ds(2 * _N + c * _L, _L)
                qx = pco[s0]
                qy = pco[s1]
                qz = pco[s2]
                bqx = bpc[s0]
                bqy = bpc[s1]
                bqz = bpc[s2]
                qn = qnb[s0]
                out = []
                for i in range(2):
                    px, py, pz, bx, by, bz, pn = row[i]
                    tk = list(st[i * 10:i * 10 + 5])
                    tv = list(st[i * 10 + 5:i * 10 + 10])
                    edot = bx * bqx + by * bqy + bz * bqz
                    ekey = (pn + qn) - 2.0 * edot
                    ex = px - qx
                    ey = py - qy
                    ez = pz - qz
                    ev = ex * ex + ey * ey + ez * ez
                    for lev in range(4):
                        cnd = ekey < tk[lev]
                        nk = jnp.minimum(tk[lev], ekey)
                        xk = jnp.maximum(tk[lev], ekey)
                        nv = jnp.where(cnd, ev, tv[lev])
                        xv = jnp.where(cnd, tv[lev], ev)
                        tk[lev] = nk
                        tv[lev] = nv
                        ekey = xk
                        ev = xv
                    cnd = ekey < tk[4]
                    tk[4] = jnp.minimum(tk[4], ekey)
                    tv[4] = jnp.where(cnd, ev, tv[4])
                    out += tk + tv
                return tuple(out)

            st = lax.fori_loop(0, _CHUNKS, chunk, init, unroll=4)
            for i in range(2):
                r = g * _L + pair * 2 + i
                for k in range(5):
                    tkb[pl.ds((r * 5 + k) * _L, _L)] = st[i * 10 + k]
                    tvb[pl.ds((r * 5 + k) * _L, _L)] = st[i * 10 + 5 + k]
        return carry

    lax.fori_loop(0, _GRP, grp16, 0)

    pltpu.sync_copy(tkb, out_tk.at[wid])
    pltpu.sync_copy(tvb, out_tv.at[wid])


_sc_knn = functools.partial(
    pl.kernel,
    out_type=(
        jax.ShapeDtypeStruct((_NW, _RPW * 5 * _L), jnp.float32),
        jax.ShapeDtypeStruct((_NW, _RPW * 5 * _L), jnp.float32),
    ),
    mesh=plsc.VectorSubcoreMesh(core_axis_name="c", subcore_axis_name="s",
                                num_cores=_NC, num_subcores=_NS),
    scratch_types=[
        pltpu.VMEM((3 * _N,), jnp.float32),
        pltpu.VMEM((3 * _N,), jnp.float32),
        pltpu.VMEM((_N,), jnp.float32),
        pltpu.VMEM((_RPW * 5 * _L,), jnp.float32),
        pltpu.VMEM((_RPW * 5 * _L,), jnp.float32),
    ],
)(_sc_body)


def _emd_body(p_ref, bp_ref, g_ref, bg_ref, out_ref):
    p = p_ref[0]
    bp = bp_ref[0]
    g = g_ref[0]
    bg = bg_ref[0]
    a2 = jnp.sum(p * p, axis=1, keepdims=True)
    b2 = jnp.sum(g * g, axis=0, keepdims=True)
    dot_k = jax.lax.dot_general(
        bp, bg, (((1,), (0,)), ((), ())),
        preferred_element_type=jnp.float32)
    key = (a2 + b2) - 2.0 * dot_k
    dot_v = jax.lax.dot_general(
        p, g, (((1,), (0,)), ((), ())),
        preferred_element_type=jnp.float32,
        precision=jax.lax.Precision.HIGHEST)
    val = (a2 + b2) - 2.0 * dot_v
    rowkey = jnp.min(key, axis=1, keepdims=True)
    rowval = jnp.min(jnp.where(key == rowkey, val, _BIG), axis=1,
                     keepdims=True)
    out_ref[pl.program_id(0), pl.program_id(1)] = jnp.sum(rowval)


def _reduce_body(pt_ref, tk_ref, tv_ref, rad_ref, emd_ref, uni_ref):
    tot = jnp.float32(0.0)
    for b in range(_B):
        pb = jnp.float32(0.0)
        for t in range(_NT):
            pb = pb + pt_ref[b, t]
        tot = tot + pb / rad_ref[b, 0]
    emd_ref[0, 0] = tot * (250.0 / (3.0 * _N * _B))

    tk = tk_ref[...]
    tv = tv_ref[...]
    k0 = jnp.min(tk, axis=1, keepdims=True)
    tk = jnp.where(tk == k0, _BIG, tk)
    acc = jnp.zeros((), jnp.float32)
    for k in range(4):
        kk = jnp.min(tk, axis=1, keepdims=True)
        sel = tk == kk
        vv = jnp.min(jnp.where(sel, tv, _BIG), axis=1, keepdims=True)
        v = jnp.maximum(vv, _EPS)
        dist = jnp.sqrt(v)
        w = jnp.exp(v * (-1.0 / (_H * _H)))
        acc = acc + jnp.sum((_RADIUS - dist) * w)
        if k < 3:
            tk = jnp.where(sel, _BIG, tk)
    uni_ref[0, 0] = acc / (_B * _N * 4.0)


def _round_bf16(x):
    u = lax.bitcast_convert_type(x, jnp.int32)
    r = (u + ((u >> 16) & 1) + 0x7FFF) & jnp.int32(-65536)
    return lax.bitcast_convert_type(r, jnp.float32)


def kernel(pred_fullpoint, gt_fullpoint, radius_data):
    pred_t = jnp.transpose(pred_fullpoint, (0, 2, 1))
    pred_f = pred_t.reshape(_B, 3 * _N)
    bpred_f = _round_bf16(pred_f)

    zpad_r = jnp.zeros((_B, _N, 5), jnp.float32)
    p_pad = jnp.concatenate([pred_fullpoint, zpad_r], axis=2)
    bp_pad = _round_bf16(p_pad)
    zpad_c = jnp.zeros((_B, 5, _N), jnp.float32)
    g_pad = jnp.concatenate(
        [jnp.transpose(gt_fullpoint, (0, 2, 1)), zpad_c], axis=1)
    bg_pad = _round_bf16(g_pad)

    tk, tv = _sc_knn(pred_f, bpred_f)

    partials = pl.pallas_call(
        _emd_body,
        grid=(_B, _NT),
        in_specs=[
            pl.BlockSpec((1, _TR, 8), lambda b, t: (b, t, 0)),
            pl.BlockSpec((1, _TR, 8), lambda b, t: (b, t, 0)),
            pl.BlockSpec((1, 8, _N), lambda b, t: (b, 0, 0)),
            pl.BlockSpec((1, 8, _N), lambda b, t: (b, 0, 0)),
        ],
        out_specs=pl.BlockSpec((_B, _NT), lambda b, t: (0, 0),
                               memory_space=pltpu.SMEM),
        out_shape=jax.ShapeDtypeStruct((_B, _NT), jnp.float32),
    )(p_pad, bp_pad, g_pad, bg_pad)
    tk_p = tk.reshape(_B * _N, 5 * _L)
    tv_p = tv.reshape(_B * _N, 5 * _L)
    rad = radius_data.reshape(_B, 1)

    emd, uni = pl.pallas_call(
        _reduce_body,
        in_specs=[
            pl.BlockSpec(memory_space=pltpu.SMEM),
            pl.BlockSpec((_B * _N, 5 * _L), lambda: (0, 0)),
            pl.BlockSpec((_B * _N, 5 * _L), lambda: (0, 0)),
            pl.BlockSpec(memory_space=pltpu.SMEM),
        ],
        out_shape=(
            jax.ShapeDtypeStruct((1, 1), jnp.float32),
            jax.ShapeDtypeStruct((1, 1), jnp.float32),
        ),
        out_specs=(
            pl.BlockSpec(memory_space=pltpu.SMEM),
            pl.BlockSpec(memory_space=pltpu.SMEM),
        ),
    )(partials, tk_p, tv_p, rad)

    return (emd[0, 0], uni[0, 0])


# speedup vs baseline: 1.0565x; 1.0565x over previous
import functools

import jax
import jax.numpy as jnp
from jax import lax
from jax.experimental import pallas as pl
from jax.experimental.pallas import tpu as pltpu
from jax.experimental.pallas import tpu_sc as plsc

_B, _N = 8, 2048
_L = 16
_NC, _NS = 2, 16
_NW = _NC * _NS
_RPW = _B * _N // _NW
_WPB = _NW // _B
_GRP = _RPW // _L
_CHUNKS = _N // _L
_TR = 256
_NT = _N // _TR
_BIG = float(3.0e38)

_RADIUS = 0.07
_H = 0.03
_EPS = 1e-12


def _sc_body(pred_hbm, bpred_hbm, out_tk, out_tv, pco, bpc, qnb, tkb, tvb):
    wid = lax.axis_index("s") * _NC + lax.axis_index("c")
    b = wid // _WPB
    base = (wid % _WPB) * _RPW

    pltpu.sync_copy(pred_hbm.at[b], pco)
    pltpu.sync_copy(bpred_hbm.at[b], bpc)

    def prep(c, carry):
        s = pl.ds(c * _L, _L)
        qx = pco[s]
        qy = pco[pl.ds(_N + c * _L, _L)]
        qz = pco[pl.ds(2 * _N + c * _L, _L)]
        qnb[s] = qx * qx + qy * qy + qz * qz
        return carry

    lax.fori_loop(0, _CHUNKS, prep, 0)

    def grp16(g, carry):
        rbase = base + g * _L
        pxc = pco[pl.ds(rbase, _L)]
        pyc = pco[pl.ds(_N + rbase, _L)]
        pzc = pco[pl.ds(2 * _N + rbase, _L)]
        bxc = bpc[pl.ds(rbase, _L)]
        byc = bpc[pl.ds(_N + rbase, _L)]
        bzc = bpc[pl.ds(2 * _N + rbase, _L)]
        pnc = qnb[pl.ds(rbase, _L)]

        for pair in range(_L // 2):
            row = []
            for i in range(2):
                l = pair * 2 + i
                row.append((
                    jnp.broadcast_to(pxc[l], (_L,)),
                    jnp.broadcast_to(pyc[l], (_L,)),
                    jnp.broadcast_to(pzc[l], (_L,)),
                    jnp.broadcast_to(bxc[l], (_L,)),
                    jnp.broadcast_to(byc[l], (_L,)),
                    jnp.broadcast_to(bzc[l], (_L,)),
                    jnp.broadcast_to(pnc[l], (_L,)),
                ))

            init = (jnp.full((_L,), _BIG, jnp.float32),) * 20

            def chunk(c, st, row=row):
                st = list(st)
                s0 = pl.ds(c * _L, _L)
                s1 = pl.ds(_N + c * _L, _L)
                s2 = pl.ds(2 * _N + c * _L, _L)
                qx = pco[s0]
                qy = pco[s1]
                qz = pco[s2]
                bqx = bpc[s0]
                bqy = bpc[s1]
                bqz = bpc[s2]
                qn = qnb[s0]
                out = []
                for i in range(2):
                    px, py, pz, bx, by, bz, pn = row[i]
                    tk = list(st[i * 10:i * 10 + 5])
                    tv = list(st[i * 10 + 5:i * 10 + 10])
                    edot = bx * bqx + by * bqy + bz * bqz
                    ekey = (pn + qn) - 2.0 * edot
                    ex = px - qx
                    ey = py - qy
                    ez = pz - qz
                    ev = ex * ex + ey * ey + ez * ez
                    for lev in range(4):
                        cnd = ekey < tk[lev]
                        nk = jnp.minimum(tk[lev], ekey)
                        xk = jnp.maximum(tk[lev], ekey)
                        nv = jnp.where(cnd, ev, tv[lev])
                        xv = jnp.where(cnd, tv[lev], ev)
                        tk[lev] = nk
                        tv[lev] = nv
                        ekey = xk
                        ev = xv
                    cnd = ekey < tk[4]
                    tk[4] = jnp.minimum(tk[4], ekey)
                    tv[4] = jnp.where(cnd, ev, tv[4])
                    out += tk + tv
                return tuple(out)

            st = lax.fori_loop(0, _CHUNKS, chunk, init, unroll=2)
            for i in range(2):
                r = g * _L + pair * 2 + i
                for k in range(5):
                    tkb[pl.ds((r * 5 + k) * _L, _L)] = st[i * 10 + k]
                    tvb[pl.ds((r * 5 + k) * _L, _L)] = st[i * 10 + 5 + k]
        return carry

    lax.fori_loop(0, _GRP, grp16, 0)

    pltpu.sync_copy(tkb, out_tk.at[wid])
    pltpu.sync_copy(tvb, out_tv.at[wid])


_sc_knn = functools.partial(
    pl.kernel,
    out_type=(
        jax.ShapeDtypeStruct((_NW, _RPW * 5 * _L), jnp.float32),
        jax.ShapeDtypeStruct((_NW, _RPW * 5 * _L), jnp.float32),
    ),
    mesh=plsc.VectorSubcoreMesh(core_axis_name="c", subcore_axis_name="s",
                                num_cores=_NC, num_subcores=_NS),
    scratch_types=[
        pltpu.VMEM((3 * _N,), jnp.float32),
        pltpu.VMEM((3 * _N,), jnp.float32),
        pltpu.VMEM((_N,), jnp.float32),
        pltpu.VMEM((_RPW * 5 * _L,), jnp.float32),
        pltpu.VMEM((_RPW * 5 * _L,), jnp.float32),
    ],
)(_sc_body)


def _emd_body(p_ref, bp_ref, g_ref, bg_ref, out_ref):
    p = p_ref[0]
    bp = bp_ref[0]
    g = g_ref[0]
    bg = bg_ref[0]
    a2 = jnp.sum(p * p, axis=1, keepdims=True)
    b2 = jnp.sum(g * g, axis=0, keepdims=True)
    dot_k = jax.lax.dot_general(
        bp, bg, (((1,), (0,)), ((), ())),
        preferred_element_type=jnp.float32)
    key = (a2 + b2) - 2.0 * dot_k
    dot_v = jax.lax.dot_general(
        p, g, (((1,), (0,)), ((), ())),
        preferred_element_type=jnp.float32,
        precision=jax.lax.Precision.HIGHEST)
    val = (a2 + b2) - 2.0 * dot_v
    rowkey = jnp.min(key, axis=1, keepdims=True)
    rowval = jnp.min(jnp.where(key == rowkey, val, _BIG), axis=1,
                     keepdims=True)
    out_ref[pl.program_id(0), pl.program_id(1)] = jnp.sum(rowval)


def _reduce_body(pt_ref, tk_ref, tv_ref, rad_ref, emd_ref, uni_ref):
    tot = jnp.float32(0.0)
    for b in range(_B):
        pb = jnp.float32(0.0)
        for t in range(_NT):
            pb = pb + pt_ref[b, t]
        tot = tot + pb / rad_ref[b, 0]
    emd_ref[0, 0] = tot * (250.0 / (3.0 * _N * _B))

    tk = tk_ref[...]
    tv = tv_ref[...]
    k0 = jnp.min(tk, axis=1, keepdims=True)
    tk = jnp.where(tk == k0, _BIG, tk)
    acc = jnp.zeros((), jnp.float32)
    for k in range(4):
        kk = jnp.min(tk, axis=1, keepdims=True)
        sel = tk == kk
        vv = jnp.min(jnp.where(sel, tv, _BIG), axis=1, keepdims=True)
        v = jnp.maximum(vv, _EPS)
        dist = jnp.sqrt(v)
        w = jnp.exp(v * (-1.0 / (_H * _H)))
        acc = acc + jnp.sum((_RADIUS - dist) * w)
        if k < 3:
            tk = jnp.where(sel, _BIG, tk)
    uni_ref[0, 0] = acc / (_B * _N * 4.0)


def _round_bf16(x):
    u = lax.bitcast_convert_type(x, jnp.int32)
    r = (u + ((u >> 16) & 1) + 0x7FFF) & jnp.int32(-65536)
    return lax.bitcast_convert_type(r, jnp.float32)


def kernel(pred_fullpoint, gt_fullpoint, radius_data):
    pred_t = jnp.transpose(pred_fullpoint, (0, 2, 1))
    pred_f = pred_t.reshape(_B, 3 * _N)
    bpred_f = _round_bf16(pred_f)

    zpad_r = jnp.zeros((_B, _N, 5), jnp.float32)
    p_pad = jnp.concatenate([pred_fullpoint, zpad_r], axis=2)
    bp_pad = _round_bf16(p_pad)
    zpad_c = jnp.zeros((_B, 5, _N), jnp.float32)
    g_pad = jnp.concatenate(
        [jnp.transpose(gt_fullpoint, (0, 2, 1)), zpad_c], axis=1)
    bg_pad = _round_bf16(g_pad)

    tk, tv = _sc_knn(pred_f, bpred_f)

    partials = pl.pallas_call(
        _emd_body,
        grid=(_B, _NT),
        in_specs=[
            pl.BlockSpec((1, _TR, 8), lambda b, t: (b, t, 0)),
            pl.BlockSpec((1, _TR, 8), lambda b, t: (b, t, 0)),
            pl.BlockSpec((1, 8, _N), lambda b, t: (b, 0, 0)),
            pl.BlockSpec((1, 8, _N), lambda b, t: (b, 0, 0)),
        ],
        out_specs=pl.BlockSpec((_B, _NT), lambda b, t: (0, 0),
                               memory_space=pltpu.SMEM),
        out_shape=jax.ShapeDtypeStruct((_B, _NT), jnp.float32),
    )(p_pad, bp_pad, g_pad, bg_pad)
    tk_p = tk.reshape(_B * _N, 5 * _L)
    tv_p = tv.reshape(_B * _N, 5 * _L)
    rad = radius_data.reshape(_B, 1)

    emd, uni = pl.pallas_call(
        _reduce_body,
        in_specs=[
            pl.BlockSpec(memory_space=pltpu.SMEM),
            pl.BlockSpec((_B * _N, 5 * _L), lambda: (0, 0)),
            pl.BlockSpec((_B * _N, 5 * _L), lambda: (0, 0)),
            pl.BlockSpec(memory_space=pltpu.SMEM),
        ],
        out_shape=(
            jax.ShapeDtypeStruct((1, 1), jnp.float32),
            jax.ShapeDtypeStruct((1, 1), jnp.float32),
        ),
        out_specs=(
            pl.BlockSpec(memory_space=pltpu.SMEM),
            pl.BlockSpec(memory_space=pltpu.SMEM),
        ),
    )(partials, tk_p, tv_p, rad)

    return (emd[0, 0], uni[0, 0])


# speedup vs baseline: 1.0900x; 1.0318x over previous
import functools

import jax
import jax.numpy as jnp
from jax import lax
from jax.experimental import pallas as pl
from jax.experimental.pallas import tpu as pltpu
from jax.experimental.pallas import tpu_sc as plsc

_B, _N = 8, 2048
_L = 16
_NC, _NS = 2, 16
_NW = _NC * _NS
_RPW = _B * _N // _NW
_WPB = _NW // _B
_GRP = _RPW // _L
_CHUNKS = _N // _L
_TR = 256
_NT = _N // _TR
_BIG = float(3.0e38)

_RADIUS = 0.07
_H = 0.03
_EPS = 1e-12


def _sc_body(pred_hbm, bpred_hbm, out_tk, out_tv, pco, bpc, qnb, tkb, tvb):
    wid = lax.axis_index("s") * _NC + lax.axis_index("c")
    b = wid // _WPB
    base = (wid % _WPB) * _RPW

    pltpu.sync_copy(pred_hbm.at[b], pco)
    pltpu.sync_copy(bpred_hbm.at[b], bpc)

    def prep(c, carry):
        s = pl.ds(c * _L, _L)
        qx = pco[s]
        qy = pco[pl.ds(_N + c * _L, _L)]
        qz = pco[pl.ds(2 * _N + c * _L, _L)]
        qnb[s] = qx * qx + qy * qy + qz * qz
        return carry

    lax.fori_loop(0, _CHUNKS, prep, 0)

    def grp16(g, carry):
        rbase = base + g * _L
        pxc = pco[pl.ds(rbase, _L)]
        pyc = pco[pl.ds(_N + rbase, _L)]
        pzc = pco[pl.ds(2 * _N + rbase, _L)]
        bxc = bpc[pl.ds(rbase, _L)]
        byc = bpc[pl.ds(_N + rbase, _L)]
        bzc = bpc[pl.ds(2 * _N + rbase, _L)]

        for pair in range(_L // 2):
            row = []
            for i in range(2):
                l = pair * 2 + i
                row.append((
                    jnp.broadcast_to(pxc[l], (_L,)),
                    jnp.broadcast_to(pyc[l], (_L,)),
                    jnp.broadcast_to(pzc[l], (_L,)),
                    jnp.broadcast_to(bxc[l], (_L,)),
                    jnp.broadcast_to(byc[l], (_L,)),
                    jnp.broadcast_to(bzc[l], (_L,)),
                ))

            init = (jnp.full((_L,), _BIG, jnp.float32),) * 20

            def chunk(c, st, row=row):
                st = list(st)
                s0 = pl.ds(c * _L, _L)
                s1 = pl.ds(_N + c * _L, _L)
                s2 = pl.ds(2 * _N + c * _L, _L)
                qx = pco[s0]
                qy = pco[s1]
                qz = pco[s2]
                bqx = bpc[s0]
                bqy = bpc[s1]
                bqz = bpc[s2]
                qn = qnb[s0]
                out = []
                for i in range(2):
                    px, py, pz, bx, by, bz = row[i]
                    tk = list(st[i * 10:i * 10 + 5])
                    tv = list(st[i * 10 + 5:i * 10 + 10])
                    edot = bx * bqx + by * bqy + bz * bqz
                    ekey = qn - 2.0 * edot
                    ex = px - qx
                    ey = py - qy
                    ez = pz - qz
                    ev = ex * ex + ey * ey + ez * ez
                    for lev in range(4):
                        cnd = ekey < tk[lev]
                        nk = jnp.minimum(tk[lev], ekey)
                        xk = jnp.maximum(tk[lev], ekey)
                        nv = jnp.where(cnd, ev, tv[lev])
                        xv = jnp.where(cnd, tv[lev], ev)
                        tk[lev] = nk
                        tv[lev] = nv
                        ekey = xk
                        ev = xv
                    cnd = ekey < tk[4]
                    tk[4] = jnp.minimum(tk[4], ekey)
                    tv[4] = jnp.where(cnd, ev, tv[4])
                    out += tk + tv
                return tuple(out)

            st = lax.fori_loop(0, _CHUNKS, chunk, init, unroll=2)
            for i in range(2):
                r = g * _L + pair * 2 + i
                for k in range(5):
                    tkb[pl.ds((r * 5 + k) * _L, _L)] = st[i * 10 + k]
                    tvb[pl.ds((r * 5 + k) * _L, _L)] = st[i * 10 + 5 + k]
        return carry

    lax.fori_loop(0, _GRP, grp16, 0)

    pltpu.sync_copy(tkb, out_tk.at[wid])
    pltpu.sync_copy(tvb, out_tv.at[wid])


_sc_knn = functools.partial(
    pl.kernel,
    out_type=(
        jax.ShapeDtypeStruct((_NW, _RPW * 5 * _L), jnp.float32),
        jax.ShapeDtypeStruct((_NW, _RPW * 5 * _L), jnp.float32),
    ),
    mesh=plsc.VectorSubcoreMesh(core_axis_name="c", subcore_axis_name="s",
                                num_cores=_NC, num_subcores=_NS),
    scratch_types=[
        pltpu.VMEM((3 * _N,), jnp.float32),
        pltpu.VMEM((3 * _N,), jnp.float32),
        pltpu.VMEM((_N,), jnp.float32),
        pltpu.VMEM((_RPW * 5 * _L,), jnp.float32),
        pltpu.VMEM((_RPW * 5 * _L,), jnp.float32),
    ],
)(_sc_body)


def _emd_body(p_ref, bp_ref, g_ref, bg_ref, out_ref):
    p = p_ref[0]
    bp = bp_ref[0]
    g = g_ref[0]
    bg = bg_ref[0]
    a2 = jnp.sum(p * p, axis=1, keepdims=True)
    b2 = jnp.sum(g * g, axis=0, keepdims=True)
    dot_k = jax.lax.dot_general(
        bp, bg, (((1,), (0,)), ((), ())),
        preferred_element_type=jnp.float32)
    key = b2 - 2.0 * dot_k
    dot_v = jax.lax.dot_general(
        p, g, (((1,), (0,)), ((), ())),
        preferred_element_type=jnp.float32,
        precision=jax.lax.Precision.HIGHEST)
    val = (a2 + b2) - 2.0 * dot_v
    rowkey = jnp.min(key, axis=1, keepdims=True)
    rowval = jnp.min(jnp.where(key == rowkey, val, _BIG), axis=1,
                     keepdims=True)
    out_ref[pl.program_id(0), pl.program_id(1)] = jnp.sum(rowval)


def _reduce_body(pt_ref, tk_ref, tv_ref, rad_ref, emd_ref, uni_ref):
    tot = jnp.float32(0.0)
    for b in range(_B):
        pb = jnp.float32(0.0)
        for t in range(_NT):
            pb = pb + pt_ref[b, t]
        tot = tot + pb / rad_ref[b, 0]
    emd_ref[0, 0] = tot * (250.0 / (3.0 * _N * _B))

    tk = tk_ref[...]
    tv = tv_ref[...]
    k0 = jnp.min(tk, axis=1, keepdims=True)
    tk = jnp.where(tk == k0, _BIG, tk)
    acc = jnp.zeros((), jnp.float32)
    for k in range(4):
        kk = jnp.min(tk, axis=1, keepdims=True)
        sel = tk == kk
        vv = jnp.min(jnp.where(sel, tv, _BIG), axis=1, keepdims=True)
        v = jnp.maximum(vv, _EPS)
        dist = jnp.sqrt(v)
        w = jnp.exp(v * (-1.0 / (_H * _H)))
        acc = acc + jnp.sum((_RADIUS - dist) * w)
        if k < 3:
            tk = jnp.where(sel, _BIG, tk)
    uni_ref[0, 0] = acc / (_B * _N * 4.0)


def _round_bf16(x):
    u = lax.bitcast_convert_type(x, jnp.int32)
    r = (u + ((u >> 16) & 1) + 0x7FFF) & jnp.int32(-65536)
    return lax.bitcast_convert_type(r, jnp.float32)


def kernel(pred_fullpoint, gt_fullpoint, radius_data):
    pred_t = jnp.transpose(pred_fullpoint, (0, 2, 1))
    pred_f = pred_t.reshape(_B, 3 * _N)
    bpred_f = _round_bf16(pred_f)

    zpad_r = jnp.zeros((_B, _N, 5), jnp.float32)
    p_pad = jnp.concatenate([pred_fullpoint, zpad_r], axis=2)
    bp_pad = _round_bf16(p_pad)
    zpad_c = jnp.zeros((_B, 5, _N), jnp.float32)
    g_pad = jnp.concatenate(
        [jnp.transpose(gt_fullpoint, (0, 2, 1)), zpad_c], axis=1)
    bg_pad = _round_bf16(g_pad)

    tk, tv = _sc_knn(pred_f, bpred_f)

    partials = pl.pallas_call(
        _emd_body,
        grid=(_B, _NT),
        in_specs=[
            pl.BlockSpec((1, _TR, 8), lambda b, t: (b, t, 0)),
            pl.BlockSpec((1, _TR, 8), lambda b, t: (b, t, 0)),
            pl.BlockSpec((1, 8, _N), lambda b, t: (b, 0, 0)),
            pl.BlockSpec((1, 8, _N), lambda b, t: (b, 0, 0)),
        ],
        out_specs=pl.BlockSpec((_B, _NT), lambda b, t: (0, 0),
                               memory_space=pltpu.SMEM),
        out_shape=jax.ShapeDtypeStruct((_B, _NT), jnp.float32),
    )(p_pad, bp_pad, g_pad, bg_pad)
    tk_p = tk.reshape(_B * _N, 5 * _L)
    tv_p = tv.reshape(_B * _N, 5 * _L)
    rad = radius_data.reshape(_B, 1)

    emd, uni = pl.pallas_call(
        _reduce_body,
        in_specs=[
            pl.BlockSpec(memory_space=pltpu.SMEM),
            pl.BlockSpec((_B * _N, 5 * _L), lambda: (0, 0)),
            pl.BlockSpec((_B * _N, 5 * _L), lambda: (0, 0)),
            pl.BlockSpec(memory_space=pltpu.SMEM),
        ],
        out_shape=(
            jax.ShapeDtypeStruct((1, 1), jnp.float32),
            jax.ShapeDtypeStruct((1, 1), jnp.float32),
        ),
        out_specs=(
            pl.BlockSpec(memory_space=pltpu.SMEM),
            pl.BlockSpec(memory_space=pltpu.SMEM),
        ),
    )(partials, tk_p, tv_p, rad)

    return (emd[0, 0], uni[0, 0])


# speedup vs baseline: 1.0904x; 1.0003x over previous
import functools

import jax
import jax.numpy as jnp
from jax import lax
from jax.experimental import pallas as pl
from jax.experimental.pallas import tpu as pltpu
from jax.experimental.pallas import tpu_sc as plsc

_B, _N = 8, 2048
_L = 16
_NC, _NS = 2, 16
_NW = _NC * _NS
_RPW = _B * _N // _NW
_WPB = _NW // _B
_GRP = _RPW // _L
_CHUNKS = _N // _L
_TR = 512
_NT = _N // _TR
_BIG = float(3.0e38)

_RADIUS = 0.07
_H = 0.03
_EPS = 1e-12


def _sc_body(pred_hbm, bpred_hbm, out_tk, out_tv, pco, bpc, qnb, tkb, tvb):
    wid = lax.axis_index("s") * _NC + lax.axis_index("c")
    b = wid // _WPB
    base = (wid % _WPB) * _RPW

    pltpu.sync_copy(pred_hbm.at[b], pco)
    pltpu.sync_copy(bpred_hbm.at[b], bpc)

    def prep(c, carry):
        s = pl.ds(c * _L, _L)
        qx = pco[s]
        qy = pco[pl.ds(_N + c * _L, _L)]
        qz = pco[pl.ds(2 * _N + c * _L, _L)]
        qnb[s] = qx * qx + qy * qy + qz * qz
        return carry

    lax.fori_loop(0, _CHUNKS, prep, 0)

    def grp16(g, carry):
        rbase = base + g * _L
        pxc = pco[pl.ds(rbase, _L)]
        pyc = pco[pl.ds(_N + rbase, _L)]
        pzc = pco[pl.ds(2 * _N + rbase, _L)]
        bxc = bpc[pl.ds(rbase, _L)]
        byc = bpc[pl.ds(_N + rbase, _L)]
        bzc = bpc[pl.ds(2 * _N + rbase, _L)]

        for pair in range(_L // 2):
            row = []
            for i in range(2):
                l = pair * 2 + i
                row.append((
                    jnp.broadcast_to(pxc[l], (_L,)),
                    jnp.broadcast_to(pyc[l], (_L,)),
                    jnp.broadcast_to(pzc[l], (_L,)),
                    jnp.broadcast_to(bxc[l], (_L,)),
                    jnp.broadcast_to(byc[l], (_L,)),
                    jnp.broadcast_to(bzc[l], (_L,)),
                ))

            init = (jnp.full((_L,), _BIG, jnp.float32),) * 20

            def chunk(c, st, row=row):
                st = list(st)
                s0 = pl.ds(c * _L, _L)
                s1 = pl.ds(_N + c * _L, _L)
                s2 = pl.ds(2 * _N + c * _L, _L)
                qx = pco[s0]
                qy = pco[s1]
                qz = pco[s2]
                bqx = bpc[s0]
                bqy = bpc[s1]
                bqz = bpc[s2]
                qn = qnb[s0]
                out = []
                for i in range(2):
                    px, py, pz, bx, by, bz = row[i]
                    tk = list(st[i * 10:i * 10 + 5])
                    tv = list(st[i * 10 + 5:i * 10 + 10])
                    edot = bx * bqx + by * bqy + bz * bqz
                    ekey = qn - 2.0 * edot
                    ex = px - qx
                    ey = py - qy
                    ez = pz - qz
                    ev = ex * ex + ey * ey + ez * ez
                    for lev in range(4):
                        cnd = ekey < tk[lev]
                        nk = jnp.minimum(tk[lev], ekey)
                        xk = jnp.maximum(tk[lev], ekey)
                        nv = jnp.where(cnd, ev, tv[lev])
                        xv = jnp.where(cnd, tv[lev], ev)
                        tk[lev] = nk
                        tv[lev] = nv
                        ekey = xk
                        ev = xv
                    cnd = ekey < tk[4]
                    tk[4] = jnp.minimum(tk[4], ekey)
                    tv[4] = jnp.where(cnd, ev, tv[4])
                    out += tk + tv
                return tuple(out)

            st = lax.fori_loop(0, _CHUNKS, chunk, init, unroll=2)
            for i in range(2):
                r = g * _L + pair * 2 + i
                for k in range(5):
                    tkb[pl.ds((r * 5 + k) * _L, _L)] = st[i * 10 + k]
                    tvb[pl.ds((r * 5 + k) * _L, _L)] = st[i * 10 + 5 + k]
        return carry

    lax.fori_loop(0, _GRP, grp16, 0)

    pltpu.sync_copy(tkb, out_tk.at[wid])
    pltpu.sync_copy(tvb, out_tv.at[wid])


_sc_knn = functools.partial(
    pl.kernel,
    out_type=(
        jax.ShapeDtypeStruct((_NW, _RPW * 5 * _L), jnp.float32),
        jax.ShapeDtypeStruct((_NW, _RPW * 5 * _L), jnp.float32),
    ),
    mesh=plsc.VectorSubcoreMesh(core_axis_name="c", subcore_axis_name="s",
                                num_cores=_NC, num_subcores=_NS),
    scratch_types=[
        pltpu.VMEM((3 * _N,), jnp.float32),
        pltpu.VMEM((3 * _N,), jnp.float32),
        pltpu.VMEM((_N,), jnp.float32),
        pltpu.VMEM((_RPW * 5 * _L,), jnp.float32),
        pltpu.VMEM((_RPW * 5 * _L,), jnp.float32),
    ],
)(_sc_body)


def _emd_body(p_ref, bp_ref, g_ref, bg_ref, out_ref):
    p = p_ref[0]
    bp = bp_ref[0]
    g = g_ref[0]
    bg = bg_ref[0]
    a2 = jnp.sum(p * p, axis=1, keepdims=True)
    b2 = jnp.sum(g * g, axis=0, keepdims=True)
    dot_k = jax.lax.dot_general(
        bp, bg, (((1,), (0,)), ((), ())),
        preferred_element_type=jnp.float32)
    key = b2 - 2.0 * dot_k
    dot_v = jax.lax.dot_general(
        p, g, (((1,), (0,)), ((), ())),
        preferred_element_type=jnp.float32,
        precision=jax.lax.Precision.HIGHEST)
    val = (a2 + b2) - 2.0 * dot_v
    rowkey = jnp.min(key, axis=1, keepdims=True)
    rowval = jnp.min(jnp.where(key == rowkey, val, _BIG), axis=1,
                     keepdims=True)
    out_ref[pl.program_id(0), pl.program_id(1)] = jnp.sum(rowval)


def _reduce_body(pt_ref, tk_ref, tv_ref, rad_ref, emd_ref, uni_ref):
    tot = jnp.float32(0.0)
    for b in range(_B):
        pb = jnp.float32(0.0)
        for t in range(_NT):
            pb = pb + pt_ref[b, t]
        tot = tot + pb / rad_ref[b, 0]
    emd_ref[0, 0] = tot * (250.0 / (3.0 * _N * _B))

    tk = tk_ref[...]
    tv = tv_ref[...]
    k0 = jnp.min(tk, axis=1, keepdims=True)
    tk = jnp.where(tk == k0, _BIG, tk)
    acc = jnp.zeros((), jnp.float32)
    for k in range(4):
        kk = jnp.min(tk, axis=1, keepdims=True)
        sel = tk == kk
        vv = jnp.min(jnp.where(sel, tv, _BIG), axis=1, keepdims=True)
        v = jnp.maximum(vv, _EPS)
        dist = jnp.sqrt(v)
        w = jnp.exp(v * (-1.0 / (_H * _H)))
        acc = acc + jnp.sum((_RADIUS - dist) * w)
        if k < 3:
            tk = jnp.where(sel, _BIG, tk)
    uni_ref[0, 0] = acc / (_B * _N * 4.0)


def _round_bf16(x):
    u = lax.bitcast_convert_type(x, jnp.int32)
    r = (u + ((u >> 16) & 1) + 0x7FFF) & jnp.int32(-65536)
    return lax.bitcast_convert_type(r, jnp.float32)


def kernel(pred_fullpoint, gt_fullpoint, radius_data):
    pred_t = jnp.transpose(pred_fullpoint, (0, 2, 1))
    pred_f = pred_t.reshape(_B, 3 * _N)
    bpred_f = _round_bf16(pred_f)

    zpad_r = jnp.zeros((_B, _N, 5), jnp.float32)
    p_pad = jnp.concatenate([pred_fullpoint, zpad_r], axis=2)
    bp_pad = _round_bf16(p_pad)
    zpad_c = jnp.zeros((_B, 5, _N), jnp.float32)
    g_pad = jnp.concatenate(
        [jnp.transpose(gt_fullpoint, (0, 2, 1)), zpad_c], axis=1)
    bg_pad = _round_bf16(g_pad)

    tk, tv = _sc_knn(pred_f, bpred_f)

    partials = pl.pallas_call(
        _emd_body,
        grid=(_B, _NT),
        in_specs=[
            pl.BlockSpec((1, _TR, 8), lambda b, t: (b, t, 0)),
            pl.BlockSpec((1, _TR, 8), lambda b, t: (b, t, 0)),
            pl.BlockSpec((1, 8, _N), lambda b, t: (b, 0, 0)),
            pl.BlockSpec((1, 8, _N), lambda b, t: (b, 0, 0)),
        ],
        out_specs=pl.BlockSpec((_B, _NT), lambda b, t: (0, 0),
                               memory_space=pltpu.SMEM),
        out_shape=jax.ShapeDtypeStruct((_B, _NT), jnp.float32),
    )(p_pad, bp_pad, g_pad, bg_pad)
    tk_p = tk.reshape(_B * _N, 5 * _L)
    tv_p = tv.reshape(_B * _N, 5 * _L)
    rad = radius_data.reshape(_B, 1)

    emd, uni = pl.pallas_call(
        _reduce_body,
        in_specs=[
            pl.BlockSpec(memory_space=pltpu.SMEM),
            pl.BlockSpec((_B * _N, 5 * _L), lambda: (0, 0)),
            pl.BlockSpec((_B * _N, 5 * _L), lambda: (0, 0)),
            pl.BlockSpec(memory_space=pltpu.SMEM),
        ],
        out_shape=(
            jax.ShapeDtypeStruct((1, 1), jnp.float32),
            jax.ShapeDtypeStruct((1, 1), jnp.float32),
        ),
        out_specs=(
            pl.BlockSpec(memory_space=pltpu.SMEM),
            pl.BlockSpec(memory_space=pltpu.SMEM),
        ),
    )(partials, tk_p, tv_p, rad)

    return (emd[0, 0], uni[0, 0])
